# baseline (speedup 1.0000x reference)
import functools

import jax
import jax.numpy as jnp
from jax import lax
from jax.experimental import pallas as pl
from jax.experimental.pallas import tpu as pltpu
from jax.experimental.pallas import tpu_sc as plsc

B = 4096
IN_DIM = 3072
Z = 4096
K = 256
L = 16
H = 512
TILE = 128

NW = 32
RW = B // NW

_PREC = jax.lax.Precision.DEFAULT


def _dot(a, b, prec=_PREC):
    return jnp.dot(a, b, precision=prec, preferred_element_type=jnp.float32)


def _tc_body(x_ref, W1_ref, b1_ref, W2_ref, b2_ref, w_ref, wsq_ref,
             G_ref, b3_ref, W4_ref, b4_ref, zt_ref, idx_ref, recon_ref):
    x = x_ref[...]
    h = jax.nn.relu(_dot(x, W1_ref[...]) + b1_ref[...])
    zt = _dot(h, W2_ref[...]) + b2_ref[...]
    zt_ref[...] = zt

    w = w_ref[...]
    wsq = wsq_ref[...]
    iota = jax.lax.broadcasted_iota(jnp.int32, (TILE, K), 1)

    ts = [_dot(zt[:, l * K:(l + 1) * K], w) for l in range(L)]
    os = []
    idxs = []
    for l in range(L):
        xl = zt[:, l * K:(l + 1) * K]
        xsq = jnp.sum(xl * xl, axis=1, keepdims=True)
        d = xsq - 2.0 * ts[l] + wsq
        dmin = jnp.min(d, axis=1, keepdims=True)
        idx = jnp.min(jnp.where(d == dmin, iota, K), axis=1, keepdims=True)
        idxs.append(idx)
        os.append((iota == idx).astype(jnp.float32))
    idx_ref[...] = jnp.concatenate(idxs, axis=1)
    O_all = jnp.concatenate(os, axis=1)
    h2 = jax.nn.relu(_dot(O_all, G_ref[...]) + b3_ref[...])
    recon_ref[...] = jax.nn.sigmoid(_dot(h2, W4_ref[...]) + b4_ref[...])


def _sc_emb_body(idx_hbm, wT_hbm, emb_hbm, wT_v, idx_v, out_v):
    c = lax.axis_index("c")
    s = lax.axis_index("s")
    wid = s * 2 + c
    pltpu.sync_copy(wT_hbm, wT_v)
    base = wid * RW
    pltpu.sync_copy(idx_hbm.at[pl.ds(base, RW)], idx_v)

    def do_row(b):
        rows = idx_v[b, :]
        col0 = jnp.zeros((L,), jnp.int32)

        def dstep(d, col):
            out_v[d, :] = plsc.load_gather(wT_v, [rows, col])
            return col + 1

        lax.fori_loop(0, K, dstep, col0, unroll=16)
        pltpu.sync_copy(out_v, emb_hbm.at[base + b])

    @pl.loop(0, RW)
    def _(b):
        do_row(b)


@functools.partial(
    pl.kernel,
    out_type=jax.ShapeDtypeStruct((B, K, L), jnp.float32),
    mesh=plsc.VectorSubcoreMesh(core_axis_name="c", subcore_axis_name="s"),
    compiler_params=pltpu.CompilerParams(use_tc_tiling_on_sc=False,
                                         needs_layout_passes=False),
    scratch_types=[
        pltpu.VMEM((K, K), jnp.float32),
        pltpu.VMEM((RW, L), jnp.int32),
        pltpu.VMEM((K, L), jnp.float32),
    ],
)
def _sc_emb(idx_hbm, wT_hbm, emb_hbm, wT_v, idx_v, out_v):
    _sc_emb_body(idx_hbm, wT_hbm, emb_hbm, wT_v, idx_v, out_v)


@functools.partial(jax.jit, static_argnums=())
def kernel(x, W1, b1, W2, b2, emb_weight, W3, b3, W4, b4):
    W1p = jnp.pad(W1, ((0, 0), (0, H - 400)))
    b1p = jnp.pad(b1, (0, H - 400)).reshape(1, H)
    W2r = jnp.pad(W2, ((0, H - 400), (0, 0)))
    W2p = W2r.reshape(H, K, L).transpose(0, 2, 1).reshape(H, Z)
    b2p = b2.reshape(K, L).T.reshape(1, Z)
    w = emb_weight
    wT = emb_weight.T
    wsq = jnp.sum(w * w, axis=0).reshape(1, K)
    W3p = jnp.pad(W3, ((0, 0), (0, H - 400)))
    G = jnp.einsum('dk,dlh->lkh', w, W3p.reshape(K, L, H),
                   precision=_PREC,
                   preferred_element_type=jnp.float32).reshape(Z, H)
    b3p = jnp.pad(b3, (0, H - 400)).reshape(1, H)
    W4p = jnp.pad(W4, ((0, H - 400), (0, 0)))
    b4r = b4.reshape(1, IN_DIM)

    n_tiles = B // TILE
    full = lambda shape: pl.BlockSpec(shape, lambda i: (0, 0))
    zt, idx_all, recon = pl.pallas_call(
        _tc_body,
        grid=(n_tiles,),
        in_specs=[
            pl.BlockSpec((TILE, IN_DIM), lambda i: (i, 0)),
            full((IN_DIM, H)), full((1, H)),
            full((H, Z)), full((1, Z)),
            full((K, K)), full((1, K)),
            full((Z, H)), full((1, H)),
            full((H, IN_DIM)), full((1, IN_DIM)),
        ],
        out_specs=[
            pl.BlockSpec((TILE, Z), lambda i: (i, 0)),
            pl.BlockSpec((TILE, L), lambda i: (i, 0)),
            pl.BlockSpec((TILE, IN_DIM), lambda i: (i, 0)),
        ],
        out_shape=[
            jax.ShapeDtypeStruct((B, Z), jnp.float32),
            jax.ShapeDtypeStruct((B, L), jnp.int32),
            jax.ShapeDtypeStruct((B, IN_DIM), jnp.float32),
        ],
    )(x, W1p, b1p, W2p, b2p, w, wsq, G, b3p, W4p, b4r)

    emb = _sc_emb(idx_all, wT)
    z_e = zt.reshape(B, L, K).transpose(0, 2, 1)
    return (recon, z_e, emb)


# speedup vs baseline: 1.0259x; 1.0259x over previous
import functools

import jax
import jax.numpy as jnp
from jax import lax
from jax.experimental import pallas as pl
from jax.experimental.pallas import tpu as pltpu
from jax.experimental.pallas import tpu_sc as plsc

B = 4096
IN_DIM = 3072
Z = 4096
K = 256
L = 16
H = 512
TILE = 128

NW = 32
RW = B // NW

_PREC = jax.lax.Precision.DEFAULT


def _dot(a, b, prec=_PREC):
    return jnp.dot(a, b, precision=prec, preferred_element_type=jnp.float32)


def _tc_body(x_ref, W1_ref, b1_ref, W2_ref, b2_ref, w_ref, wsq_ref,
             G_ref, b3_ref, W4_ref, b4_ref, zt_ref, idx_ref, recon_ref):
    x = x_ref[...]
    h = jax.nn.relu(_dot(x, W1_ref[...]) + b1_ref[...])
    zt = _dot(h, W2_ref[...]) + b2_ref[...]
    zt_ref[...] = zt

    w = w_ref[...]
    wsq = wsq_ref[...]
    iota = jax.lax.broadcasted_iota(jnp.int32, (TILE, K), 1)

    ts = [_dot(zt[:, l * K:(l + 1) * K], w) for l in range(L)]
    os = []
    idxs = []
    for l in range(L):
        xl = zt[:, l * K:(l + 1) * K]
        xsq = jnp.sum(xl * xl, axis=1, keepdims=True)
        d = xsq - 2.0 * ts[l] + wsq
        dmin = jnp.min(d, axis=1, keepdims=True)
        idx = jnp.min(jnp.where(d == dmin, iota, K), axis=1, keepdims=True)
        idxs.append(idx)
        os.append((iota == idx).astype(jnp.float32))
    idx_ref[...] = jnp.concatenate(idxs, axis=1)
    O_all = jnp.concatenate(os, axis=1)
    h2 = jax.nn.relu(_dot(O_all, G_ref[...]) + b3_ref[...])
    recon_ref[...] = jax.nn.sigmoid(_dot(h2, W4_ref[...]) + b4_ref[...])


GR = 4
NG = RW // GR


def _sc_emb_body(idx_hbm, wT_hbm, emb_hbm, wT_v, idx_v, out_v, sem0, sem1):
    c = lax.axis_index("c")
    s = lax.axis_index("s")
    wid = s * 2 + c
    pltpu.sync_copy(wT_hbm, wT_v)
    base = wid * RW
    pltpu.sync_copy(idx_hbm.at[pl.ds(base, RW)], idx_v)
    sems = (sem0, sem1)

    @pl.loop(0, NG, step=2)
    def _(g):
        for half in range(2):
            gg = g + half
            sem = sems[half]

            @pl.when(gg >= 2)
            def _():
                pltpu.make_async_copy(
                    out_v.at[half], emb_hbm.at[pl.ds(base, GR)], sem).wait()

            rows = [idx_v[gg * GR + r, :] for r in range(GR)]
            col0 = jnp.zeros((L,), jnp.int32)

            def dstep(d, col, half=half, rows=rows):
                for r in range(GR):
                    out_v[half, r, d, :] = plsc.load_gather(wT_v, [rows[r], col])
                return col + 1

            lax.fori_loop(0, K, dstep, col0, unroll=8)
            pltpu.async_copy(
                out_v.at[half], emb_hbm.at[pl.ds(base + gg * GR, GR)], sem)

    for half in range(2):
        pltpu.make_async_copy(
            out_v.at[half], emb_hbm.at[pl.ds(base, GR)], sems[half]).wait()


@functools.partial(
    pl.kernel,
    out_type=jax.ShapeDtypeStruct((B, K, L), jnp.float32),
    mesh=plsc.VectorSubcoreMesh(core_axis_name="c", subcore_axis_name="s"),
    compiler_params=pltpu.CompilerParams(use_tc_tiling_on_sc=False,
                                         needs_layout_passes=False),
    scratch_types=[
        pltpu.VMEM((K, K), jnp.float32),
        pltpu.VMEM((RW, L), jnp.int32),
        pltpu.VMEM((2, GR, K, L), jnp.float32),
        pltpu.SemaphoreType.DMA,
        pltpu.SemaphoreType.DMA,
    ],
)
def _sc_emb(idx_hbm, wT_hbm, emb_hbm, wT_v, idx_v, out_v, sem0, sem1):
    _sc_emb_body(idx_hbm, wT_hbm, emb_hbm, wT_v, idx_v, out_v, sem0, sem1)


@functools.partial(jax.jit, static_argnums=())
def kernel(x, W1, b1, W2, b2, emb_weight, W3, b3, W4, b4):
    W1p = jnp.pad(W1, ((0, 0), (0, H - 400)))
    b1p = jnp.pad(b1, (0, H - 400)).reshape(1, H)
    W2r = jnp.pad(W2, ((0, H - 400), (0, 0)))
    W2p = W2r.reshape(H, K, L).transpose(0, 2, 1).reshape(H, Z)
    b2p = b2.reshape(K, L).T.reshape(1, Z)
    w = emb_weight
    wT = emb_weight.T
    wsq = jnp.sum(w * w, axis=0).reshape(1, K)
    W3p = jnp.pad(W3, ((0, 0), (0, H - 400)))
    G = jnp.einsum('dk,dlh->lkh', w, W3p.reshape(K, L, H),
                   precision=_PREC,
                   preferred_element_type=jnp.float32).reshape(Z, H)
    b3p = jnp.pad(b3, (0, H - 400)).reshape(1, H)
    W4p = jnp.pad(W4, ((0, H - 400), (0, 0)))
    b4r = b4.reshape(1, IN_DIM)

    n_tiles = B // TILE
    full = lambda shape: pl.BlockSpec(shape, lambda i: (0, 0))
    zt, idx_all, recon = pl.pallas_call(
        _tc_body,
        grid=(n_tiles,),
        in_specs=[
            pl.BlockSpec((TILE, IN_DIM), lambda i: (i, 0)),
            full((IN_DIM, H)), full((1, H)),
            full((H, Z)), full((1, Z)),
            full((K, K)), full((1, K)),
            full((Z, H)), full((1, H)),
            full((H, IN_DIM)), full((1, IN_DIM)),
        ],
        out_specs=[
            pl.BlockSpec((TILE, Z), lambda i: (i, 0)),
            pl.BlockSpec((TILE, L), lambda i: (i, 0)),
            pl.BlockSpec((TILE, IN_DIM), lambda i: (i, 0)),
        ],
        out_shape=[
            jax.ShapeDtypeStruct((B, Z), jnp.float32),
            jax.ShapeDtypeStruct((B, L), jnp.int32),
            jax.ShapeDtypeStruct((B, IN_DIM), jnp.float32),
        ],
    )(x, W1p, b1p, W2p, b2p, w, wsq, G, b3p, W4p, b4r)

    emb = _sc_emb(idx_all, wT)
    z_e = zt.reshape(B, L, K).transpose(0, 2, 1)
    return (recon, z_e, emb)


# speedup vs baseline: 3.2077x; 3.1267x over previous
import functools

import jax
import jax.numpy as jnp
from jax.experimental import pallas as pl
from jax.experimental.pallas import tpu as pltpu

B = 4096
IN_DIM = 3072
Z = 4096
K = 256
L = 16
H = 400
TILE = 256

_PREC = jax.lax.Precision.DEFAULT
_EXACT = jax.lax.Precision.HIGHEST


def _dot(a, b, prec=_PREC):
    return jnp.dot(a, b, precision=prec, preferred_element_type=jnp.float32)


def _body(x_ref, W1_ref, b1_ref, W2_ref, b2_ref, w_ref, wT_ref, wsq_ref,
          G_ref, b3_ref, W4_ref, b4_ref, zt_ref, emb_ref, recon_ref):
    x = x_ref[...]
    h = jax.nn.relu(_dot(x, W1_ref[...]) + b1_ref[...])
    zt = _dot(h, W2_ref[...]) + b2_ref[...]
    zt_ref[...] = zt

    w = w_ref[...]
    wsq = wsq_ref[...]
    iota = jax.lax.broadcasted_iota(jnp.int32, (TILE, K), 1)

    ts = [_dot(zt[:, l * K:(l + 1) * K], w) for l in range(L)]
    os = []
    for l in range(L):
        xl = zt[:, l * K:(l + 1) * K]
        xsq = jnp.sum(xl * xl, axis=1, keepdims=True)
        d = xsq - 2.0 * ts[l] + wsq
        dmin = jnp.min(d, axis=1, keepdims=True)
        idx = jnp.min(jnp.where(d == dmin, iota, K), axis=1, keepdims=True)
        os.append((iota == idx).astype(jnp.float32))
    O_all = jnp.concatenate(os, axis=1)
    acc = _dot(O_all, G_ref[...]) + b3_ref[...]
    wT = wT_ref[...]
    for l in range(L):
        emb_ref[:, l * K:(l + 1) * K] = _dot(os[l], wT, prec=_EXACT)
    h2 = jax.nn.relu(acc)
    recon_ref[...] = jax.nn.sigmoid(_dot(h2, W4_ref[...]) + b4_ref[...])


@functools.partial(jax.jit, static_argnums=())
def kernel(x, W1, b1, W2, b2, emb_weight, W3, b3, W4, b4):
    W1p = W1
    b1p = b1.reshape(1, H)
    W2p = W2.reshape(H, K, L).transpose(0, 2, 1).reshape(H, Z)
    b2p = b2.reshape(K, L).T.reshape(1, Z)
    w = emb_weight
    wT = emb_weight.T
    wsq = jnp.sum(w * w, axis=0).reshape(1, K)
    G = jnp.einsum('dk,dlh->lkh', w, W3.reshape(K, L, H),
                   precision=_PREC,
                   preferred_element_type=jnp.float32).reshape(Z, H)
    b3p = b3.reshape(1, H)
    W4p = W4
    b4r = b4.reshape(1, IN_DIM)

    n_tiles = B // TILE
    full = lambda shape: pl.BlockSpec(shape, lambda i: (0, 0))
    zt, emb_flat, recon = pl.pallas_call(
        _body,
        grid=(n_tiles,),
        compiler_params=pltpu.CompilerParams(
            vmem_limit_bytes=62 * 1024 * 1024),
        in_specs=[
            pl.BlockSpec((TILE, IN_DIM), lambda i: (i, 0)),
            full((IN_DIM, H)), full((1, H)),
            full((H, Z)), full((1, Z)),
            full((K, K)), full((K, K)), full((1, K)),
            full((Z, H)), full((1, H)),
            full((H, IN_DIM)), full((1, IN_DIM)),
        ],
        out_specs=[
            pl.BlockSpec((TILE, Z), lambda i: (i, 0)),
            pl.BlockSpec((TILE, Z), lambda i: (i, 0)),
            pl.BlockSpec((TILE, IN_DIM), lambda i: (i, 0)),
        ],
        out_shape=[
            jax.ShapeDtypeStruct((B, Z), jnp.float32),
            jax.ShapeDtypeStruct((B, Z), jnp.float32),
            jax.ShapeDtypeStruct((B, IN_DIM), jnp.float32),
        ],
    )(x, W1p, b1p, W2p, b2p, w, wT, wsq, G, b3p, W4p, b4r)

    z_e = zt.reshape(B, L, K).transpose(0, 2, 1)
    emb = emb_flat.reshape(B, L, K).transpose(0, 2, 1)
    return (recon, z_e, emb)


# speedup vs baseline: 3.5052x; 1.0927x over previous
import functools

import jax
import jax.numpy as jnp
from jax.experimental import pallas as pl
from jax.experimental.pallas import tpu as pltpu

B = 4096
IN_DIM = 3072
Z = 4096
K = 256
L = 16
H = 400
TILE = 256

_PREC = jax.lax.Precision.DEFAULT


def _dot(a, b, prec=_PREC):
    return jnp.dot(a, b, precision=prec, preferred_element_type=jnp.float32)


def _body(x_ref, W1_ref, b1_ref, W2_ref, b2_ref, w_ref, wT_ref, wsq_ref,
          G_ref, b3_ref, W4_ref, b4_ref, zt_ref, emb_ref, recon_ref):
    x = x_ref[...]
    h = jax.nn.relu(_dot(x, W1_ref[...]) + b1_ref[...])
    zt = _dot(h, W2_ref[...]) + b2_ref[...]
    zt_ref[...] = zt

    w = w_ref[...]
    wsq = wsq_ref[...]
    iota = jax.lax.broadcasted_iota(jnp.int32, (TILE, K), 1)

    ts = [_dot(zt[:, l * K:(l + 1) * K], w) for l in range(L)]
    os = []
    for l in range(L):
        xl = zt[:, l * K:(l + 1) * K]
        xsq = jnp.sum(xl * xl, axis=1, keepdims=True)
        d = xsq - 2.0 * ts[l] + wsq
        dmin = jnp.min(d, axis=1, keepdims=True)
        idx = jnp.min(jnp.where(d == dmin, iota, K), axis=1, keepdims=True)
        os.append((iota == idx).astype(jnp.float32))
    O_all = jnp.concatenate(os, axis=1)
    acc = _dot(O_all, G_ref[...]) + b3_ref[...]
    wT3 = wT_ref[...]
    for l in range(L):
        em = _dot(os[l], wT3)
        emb_ref[:, l * K:(l + 1) * K] = (
            em[:, :K] + em[:, K:2 * K] + em[:, 2 * K:])
    h2 = jax.nn.relu(acc)
    recon_ref[...] = jax.nn.sigmoid(_dot(h2, W4_ref[...]) + b4_ref[...])


@functools.partial(jax.jit, static_argnums=())
def kernel(x, W1, b1, W2, b2, emb_weight, W3, b3, W4, b4):
    W1p = W1
    b1p = b1.reshape(1, H)
    W2p = W2.reshape(H, K, L).transpose(0, 2, 1).reshape(H, Z)
    b2p = b2.reshape(K, L).T.reshape(1, Z)
    w = emb_weight
    wT = emb_weight.T
    wT_hi = wT.astype(jnp.bfloat16).astype(jnp.float32)
    r = wT - wT_hi
    wT_mid = r.astype(jnp.bfloat16).astype(jnp.float32)
    wT3 = jnp.concatenate([wT_hi, wT_mid, r - wT_mid], axis=1)
    wsq = jnp.sum(w * w, axis=0).reshape(1, K)
    G = jnp.einsum('dk,dlh->lkh', w, W3.reshape(K, L, H),
                   precision=_PREC,
                   preferred_element_type=jnp.float32).reshape(Z, H)
    b3p = b3.reshape(1, H)
    W4p = W4
    b4r = b4.reshape(1, IN_DIM)

    n_tiles = B // TILE
    full = lambda shape: pl.BlockSpec(shape, lambda i: (0, 0))
    zt, emb_flat, recon = pl.pallas_call(
        _body,
        grid=(n_tiles,),
        compiler_params=pltpu.CompilerParams(
            vmem_limit_bytes=62 * 1024 * 1024),
        in_specs=[
            pl.BlockSpec((TILE, IN_DIM), lambda i: (i, 0)),
            full((IN_DIM, H)), full((1, H)),
            full((H, Z)), full((1, Z)),
            full((K, K)), full((K, 3 * K)), full((1, K)),
            full((Z, H)), full((1, H)),
            full((H, IN_DIM)), full((1, IN_DIM)),
        ],
        out_specs=[
            pl.BlockSpec((TILE, Z), lambda i: (i, 0)),
            pl.BlockSpec((TILE, Z), lambda i: (i, 0)),
            pl.BlockSpec((TILE, IN_DIM), lambda i: (i, 0)),
        ],
        out_shape=[
            jax.ShapeDtypeStruct((B, Z), jnp.float32),
            jax.ShapeDtypeStruct((B, Z), jnp.float32),
            jax.ShapeDtypeStruct((B, IN_DIM), jnp.float32),
        ],
    )(x, W1p, b1p, W2p, b2p, w, wT3, wsq, G, b3p, W4p, b4r)

    z_e = zt.reshape(B, L, K).transpose(0, 2, 1)
    emb = emb_flat.reshape(B, L, K).transpose(0, 2, 1)
    return (recon, z_e, emb)


# speedup vs baseline: 3.7341x; 1.0653x over previous
import functools

import jax
import jax.numpy as jnp
from jax.experimental import pallas as pl
from jax.experimental.pallas import tpu as pltpu

B = 4096
IN_DIM = 3072
Z = 4096
K = 256
L = 16
H = 400
TILE = 256

_PREC = jax.lax.Precision.DEFAULT


def _dot(a, b):
    return jnp.dot(a, b, precision=_PREC, preferred_element_type=jnp.float32)


def _enc_body(x_ref, W1_ref, b1_ref, W2_ref, b2_ref, zt_ref):
    h = jax.nn.relu(_dot(x_ref[...], W1_ref[...]) + b1_ref[...])
    zt_ref[...] = _dot(h, W2_ref[...]) + b2_ref[...]


def _vq_body(zt_ref, w_ref, wT_ref, wsq_ref, emb_ref, idx_ref):
    zt = zt_ref[...]
    w = w_ref[...]
    wsq = wsq_ref[...]
    iota = jax.lax.broadcasted_iota(jnp.int32, (TILE, K), 1)
    ts = [_dot(zt[:, l * K:(l + 1) * K], w) for l in range(L)]
    os = []
    idxs = []
    for l in range(L):
        xl = zt[:, l * K:(l + 1) * K]
        xsq = jnp.sum(xl * xl, axis=1, keepdims=True)
        d = xsq - 2.0 * ts[l] + wsq
        dmin = jnp.min(d, axis=1, keepdims=True)
        idx = jnp.min(jnp.where(d == dmin, iota, K), axis=1, keepdims=True)
        idxs.append(idx)
        os.append((iota == idx).astype(jnp.float32))
    idx_ref[...] = jnp.concatenate(idxs, axis=1)
    wT3 = wT_ref[...]
    for l in range(L):
        em = _dot(os[l], wT3)
        emb_ref[:, l * K:(l + 1) * K] = (
            em[:, :K] + em[:, K:2 * K] + em[:, 2 * K:])


def _dec_body(idx_ref, G_ref, b3_ref, W4_ref, b4_ref, recon_ref):
    iota = jax.lax.broadcasted_iota(jnp.int32, (TILE, K), 1)
    idxs = idx_ref[...]
    os = [(iota == idxs[:, l:l + 1]).astype(jnp.float32) for l in range(L)]
    O_all = jnp.concatenate(os, axis=1)
    h2 = jax.nn.relu(_dot(O_all, G_ref[...]) + b3_ref[...])
    recon_ref[...] = jax.nn.sigmoid(_dot(h2, W4_ref[...]) + b4_ref[...])


@functools.partial(jax.jit, static_argnums=())
def kernel(x, W1, b1, W2, b2, emb_weight, W3, b3, W4, b4):
    W2p = W2.reshape(H, K, L).transpose(0, 2, 1).reshape(H, Z)
    b2p = b2.reshape(K, L).T.reshape(1, Z)
    w = emb_weight
    wT = emb_weight.T
    wT_hi = wT.astype(jnp.bfloat16).astype(jnp.float32)
    r = wT - wT_hi
    wT_mid = r.astype(jnp.bfloat16).astype(jnp.float32)
    wT3 = jnp.concatenate([wT_hi, wT_mid, r - wT_mid], axis=1)
    wsq = jnp.sum(w * w, axis=0).reshape(1, K)
    G = jnp.einsum('dk,dlh->lkh', w, W3.reshape(K, L, H),
                   precision=_PREC,
                   preferred_element_type=jnp.float32).reshape(Z, H)

    n_tiles = B // TILE
    full = lambda shape: pl.BlockSpec(shape, lambda i: (0,) * len(shape))
    row = lambda shape: pl.BlockSpec(shape, lambda i: (i, 0))
    params = pltpu.CompilerParams(vmem_limit_bytes=62 * 1024 * 1024)

    zt = pl.pallas_call(
        _enc_body,
        grid=(n_tiles,),
        compiler_params=params,
        in_specs=[row((TILE, IN_DIM)), full((IN_DIM, H)), full((1, H)),
                  full((H, Z)), full((1, Z))],
        out_specs=row((TILE, Z)),
        out_shape=jax.ShapeDtypeStruct((B, Z), jnp.float32),
    )(x, W1, b1.reshape(1, H), W2p, b2p)

    emb_flat, idx_all = pl.pallas_call(
        _vq_body,
        grid=(n_tiles,),
        compiler_params=params,
        in_specs=[row((TILE, Z)), full((K, K)), full((K, 3 * K)),
                  full((1, K))],
        out_specs=[row((TILE, Z)), row((TILE, L))],
        out_shape=[jax.ShapeDtypeStruct((B, Z), jnp.float32),
                   jax.ShapeDtypeStruct((B, L), jnp.int32)],
    )(zt, w, wT3, wsq)

    recon = pl.pallas_call(
        _dec_body,
        grid=(n_tiles,),
        compiler_params=params,
        in_specs=[row((TILE, L)), full((Z, H)), full((1, H)),
                  full((H, IN_DIM)), full((1, IN_DIM))],
        out_specs=row((TILE, IN_DIM)),
        out_shape=jax.ShapeDtypeStruct((B, IN_DIM), jnp.float32),
    )(idx_all, G, b3.reshape(1, H), W4, b4.reshape(1, IN_DIM))

    z_e = zt.reshape(B, L, K).transpose(0, 2, 1)
    emb = emb_flat.reshape(B, L, K).transpose(0, 2, 1)
    return (recon, z_e, emb)
